# baseline (speedup 1.0000x reference)
import jax
import jax.numpy as jnp
import numpy as np
from jax import lax
from jax.experimental import pallas as pl
from jax.experimental.pallas import tpu as pltpu
from jax.experimental.pallas import tpu_sc as plsc

B, C, D, H, W = 2, 128, 32, 64, 64
DHW = D * H * W
HW = H * W
PD = PH = PW = 7
CELLS = PD * PH * PW
NR = 1000
NR_PAD = 1024
NW = 32
ROIS_PER_W = NR_PAD // NW
GROUPS = ROIS_PER_W // 16
ROWS_PER_W = ROIS_PER_W * CELLS
CHUNK = 112
NCHUNK = ROWS_PER_W // CHUNK

_LIN_T = [np.float32(np.float32(k) / np.float32(6)) for k in range(6)]
_LIN_OMT = [np.float32(np.float32(1) - t) for t in _LIN_T]


def _grid_indices(a, b, hi):
    af = jnp.clip(a, 0.0, float(hi - 1))
    bf = jnp.clip(b, 0.0, float(hi - 1))
    out = []
    for k in range(7):
        if k == 6:
            g = bf
        else:
            g = af * _LIN_OMT[k] + bf * _LIN_T[k]
        out.append(jnp.clip(g.astype(jnp.int32), 0, hi - 1))
    return out


def _roi_gather_body(table, rois, out, roi_v, idx_v, buf0, buf1, sem0, sem1):
    wid = lax.axis_index("s") * 2 + lax.axis_index("c")

    pltpu.sync_copy(rois.at[pl.ds(wid * ROIS_PER_W * 8, ROIS_PER_W * 8)], roi_v)

    lanes = lax.iota(jnp.int32, 16)
    for g in range(GROUPS):
        base_addr = g * 128 + lanes * 8

        def fld(f):
            return plsc.load_gather(roi_v, [base_addr + f])

        bi = jnp.clip(fld(0).astype(jnp.int32), 0, B - 1)
        x1, y1, z1 = fld(1), fld(2), fld(3)
        x2, y2, z2 = fld(4), fld(5), fld(6)
        ix = _grid_indices(x1, x2, W)
        iy = _grid_indices(y1, y2, H)
        iz = _grid_indices(z1, z2, D)

        rowb = bi * DHW
        pos_base = g * 16 * CELLS + lanes * CELLS
        cell = 0
        for k3 in range(PD):
            t3 = rowb + iz[k3] * HW
            for k2 in range(PH):
                t32 = t3 + iy[k2] * W
                for k1 in range(PW):
                    plsc.store_scatter(idx_v, [pos_base + cell], t32 + ix[k1])
                    cell += 1

    out_base = wid * ROWS_PER_W

    def gather(c, buf, sem):
        return pltpu.async_copy(table.at[idx_v.at[pl.ds(c * CHUNK, CHUNK)]], buf, sem)

    def flush(c, buf):
        pltpu.sync_copy(buf, out.at[pl.ds(out_base + c * CHUNK, CHUNK)])

    def pair(i, _):
        c0 = 2 * i
        c1 = c0 + 1
        cp0 = gather(c0, buf0, sem0)
        cp1 = gather(c1, buf1, sem1)
        cp0.wait()
        flush(c0, buf0)
        cp1.wait()
        flush(c1, buf1)
        return _

    lax.fori_loop(0, NCHUNK // 2, pair, None)


_mesh = plsc.VectorSubcoreMesh(core_axis_name="c", subcore_axis_name="s")

_roi_gather = pl.kernel(
    _roi_gather_body,
    out_type=jax.ShapeDtypeStruct((NR_PAD * CELLS, C), jnp.float32),
    mesh=_mesh,
    scratch_types=[
        pltpu.VMEM((ROIS_PER_W * 8,), jnp.float32),
        pltpu.VMEM((ROWS_PER_W,), jnp.int32),
        pltpu.VMEM((CHUNK, C), jnp.float32),
        pltpu.VMEM((CHUNK, C), jnp.float32),
        pltpu.SemaphoreType.DMA,
        pltpu.SemaphoreType.DMA,
    ],
    compiler_params=pltpu.CompilerParams(needs_layout_passes=False),
)


@jax.jit
def kernel(features, rois):
    table = jnp.transpose(features, (0, 2, 3, 4, 1)).reshape(B * DHW, C)
    rois_p = jnp.pad(rois, ((0, NR_PAD - NR), (0, 1))).reshape(-1)
    gathered = _roi_gather(table, rois_p)
    pooled = gathered[: NR * CELLS].reshape(NR, PD, PH, PW, C)
    return jnp.transpose(pooled, (0, 4, 1, 2, 3))
